# baseline (speedup 1.0000x reference)
import functools

import jax
import jax.numpy as jnp
from jax import lax
from jax.experimental import pallas as pl
from jax.experimental.pallas import tpu as pltpu
from jax.experimental.pallas import tpu_sc as plsc


@functools.lru_cache(maxsize=None)
def _build_gather(b, l, vocab, d):
    info = plsc.get_sparse_core_info()
    nw = info.num_cores * info.num_subcores
    rpw = b // nw
    rpc = 4
    chunk = rpc * l
    n_chunks = rpw // rpc
    assert b % nw == 0 and rpw % rpc == 0 and n_chunks % 2 == 0

    mesh = plsc.VectorSubcoreMesh(core_axis_name="c", subcore_axis_name="s")

    @functools.partial(
        pl.kernel,
        mesh=mesh,
        out_type=jax.ShapeDtypeStruct((b, l, d), jnp.float32),
        scratch_types=[
            pltpu.VMEM((chunk,), jnp.int32),
            pltpu.VMEM((chunk,), jnp.int32),
            pltpu.VMEM((chunk, d), jnp.float32),
            pltpu.VMEM((chunk, d), jnp.float32),
            pltpu.SemaphoreType.DMA,
            pltpu.SemaphoreType.DMA,
            pltpu.SemaphoreType.DMA,
            pltpu.SemaphoreType.DMA,
        ],
        compiler_params=pltpu.CompilerParams(use_tc_tiling_on_sc=False),
    )
    def gather(x_hbm, table_hbm, out_hbm, idx0, idx1, rows0, rows1,
               sg0, sg1, sw0, sw1):
        wid = lax.axis_index("s") * info.num_cores + lax.axis_index("c")
        base = wid * rpw
        bufs = ((idx0, rows0, sg0, sw0), (idx1, rows1, sg1, sw1))

        def start_gather(g, idx_v, rows_v, sg):
            row0 = base + g * rpc
            for j in range(rpc):
                pltpu.sync_copy(x_hbm.at[row0 + j], idx_v.at[pl.ds(j * l, l)])
            pltpu.async_copy(table_hbm.at[idx_v], rows_v, sg)

        def wait_gather(idx_v, rows_v, sg):
            pltpu.make_async_copy(table_hbm.at[idx_v], rows_v, sg).wait()

        def start_wb(g, rows_v, sw):
            row0 = base + g * rpc
            for j in range(rpc):
                pltpu.async_copy(rows_v.at[pl.ds(j * l, l), :],
                                 out_hbm.at[row0 + j], sw)

        def wait_wb(g, rows_v, sw):
            row0 = base + g * rpc
            for j in range(rpc):
                pltpu.make_async_copy(rows_v.at[pl.ds(j * l, l), :],
                                      out_hbm.at[row0 + j], sw).wait()

        for bi, (idx_v, rows_v, sg, sw) in enumerate(bufs):
            start_gather(bi, idx_v, rows_v, sg)

        def body(i, carry):
            g0 = 2 * i
            for bi, (idx_v, rows_v, sg, sw) in enumerate(bufs):
                wait_gather(idx_v, rows_v, sg)
                start_wb(g0 + bi, rows_v, sw)
            for bi, (idx_v, rows_v, sg, sw) in enumerate(bufs):
                wait_wb(g0 + bi, rows_v, sw)
                start_gather(g0 + bi + 2, idx_v, rows_v, sg)
            return carry

        lax.fori_loop(0, n_chunks // 2 - 1, body, 0)

        gl = n_chunks - 2
        for bi, (idx_v, rows_v, sg, sw) in enumerate(bufs):
            wait_gather(idx_v, rows_v, sg)
            start_wb(gl + bi, rows_v, sw)
        for bi, (idx_v, rows_v, sg, sw) in enumerate(bufs):
            wait_wb(gl + bi, rows_v, sw)

    return gather


def kernel(x, table):
    b, l = x.shape
    vocab, d = table.shape
    idx = x.astype(jnp.int32)
    return _build_gather(b, l, vocab, d)(idx, table)


# speedup vs baseline: 1.2373x; 1.2373x over previous
import functools

import jax
import jax.numpy as jnp
from jax import lax
from jax.experimental import pallas as pl
from jax.experimental.pallas import tpu as pltpu
from jax.experimental.pallas import tpu_sc as plsc


@functools.lru_cache(maxsize=None)
def _build_gather(b, l, vocab, dp):
    info = plsc.get_sparse_core_info()
    nw = info.num_cores * info.num_subcores
    n = b * l
    bpw = n // nw
    rpc = 2
    chunk = rpc * l
    n_chunks = bpw // chunk
    assert n % nw == 0 and bpw % chunk == 0 and n_chunks % 2 == 0

    mesh = plsc.VectorSubcoreMesh(core_axis_name="c", subcore_axis_name="s")

    @functools.partial(
        pl.kernel,
        mesh=mesh,
        out_type=jax.ShapeDtypeStruct((b, l, dp), jnp.float32),
        scratch_types=[
            pltpu.VMEM((chunk,), jnp.int32),
            pltpu.VMEM((chunk,), jnp.int32),
            pltpu.VMEM((chunk, dp), jnp.float32),
            pltpu.VMEM((chunk, dp), jnp.float32),
            pltpu.SemaphoreType.DMA,
            pltpu.SemaphoreType.DMA,
            pltpu.SemaphoreType.DMA,
            pltpu.SemaphoreType.DMA,
        ],
    )
    def gather(idx_hbm, table_hbm, out_hbm, idx0, idx1, rows0, rows1,
               sg0, sg1, sw0, sw1):
        wid = lax.axis_index("s") * info.num_cores + lax.axis_index("c")
        base = wid * bpw
        row_base = wid * (bpw // l)
        bufs = ((idx0, rows0, sg0, sw0), (idx1, rows1, sg1, sw1))

        def start_gather(g, idx_v, rows_v, sg):
            pltpu.sync_copy(idx_hbm.at[pl.ds(base + g * chunk, chunk)], idx_v)
            pltpu.async_copy(table_hbm.at[idx_v], rows_v, sg)

        def wait_gather(idx_v, rows_v, sg):
            pltpu.make_async_copy(table_hbm.at[idx_v], rows_v, sg).wait()

        def start_wb(g, rows_v, sw):
            row0 = row_base + g * rpc
            for j in range(rpc):
                pltpu.async_copy(rows_v.at[pl.ds(j * l, l), :],
                                 out_hbm.at[row0 + j], sw)

        def wait_wb(g, rows_v, sw):
            row0 = row_base + g * rpc
            for j in range(rpc):
                pltpu.make_async_copy(rows_v.at[pl.ds(j * l, l), :],
                                      out_hbm.at[row0 + j], sw).wait()

        for bi, (idx_v, rows_v, sg, sw) in enumerate(bufs):
            start_gather(bi, idx_v, rows_v, sg)

        def body(i, carry):
            g0 = 2 * i
            for bi, (idx_v, rows_v, sg, sw) in enumerate(bufs):
                wait_gather(idx_v, rows_v, sg)
                start_wb(g0 + bi, rows_v, sw)
            for bi, (idx_v, rows_v, sg, sw) in enumerate(bufs):
                wait_wb(g0 + bi, rows_v, sw)
                start_gather(g0 + bi + 2, idx_v, rows_v, sg)
            return carry

        lax.fori_loop(0, n_chunks // 2 - 1, body, 0)

        gl = n_chunks - 2
        for bi, (idx_v, rows_v, sg, sw) in enumerate(bufs):
            wait_gather(idx_v, rows_v, sg)
            start_wb(gl + bi, rows_v, sw)
        for bi, (idx_v, rows_v, sg, sw) in enumerate(bufs):
            wait_wb(gl + bi, rows_v, sw)

    return gather


def kernel(x, table):
    b, l = x.shape
    vocab, d = table.shape
    idx = x.astype(jnp.int32).reshape(-1)
    tpad = jnp.pad(table, ((0, 0), (0, 128 - d)))
    out = _build_gather(b, l, vocab, 128)(idx, tpad)
    return out[:, :, :d]
